# baseline (speedup 1.0000x reference)
import functools

import numpy as np
import jax
import jax.numpy as jnp
from jax import lax
from jax.experimental import pallas as pl
from jax.experimental.pallas import tpu as pltpu
from jax.experimental.pallas import tpu_sc as plsc

B, NP1, K = 64, 4096, 32
NC, NS, L = 2, 16, 16
NW = NC * NS
BPW = B // NW
J_CH = 256
N_CH = NP1 // J_CH


@functools.lru_cache(maxsize=1)
def _sc_neighbor_sums():
    mesh = plsc.VectorSubcoreMesh(
        core_axis_name="c", subcore_axis_name="s", num_cores=NC, num_subcores=NS
    )

    @functools.partial(
        pl.kernel,
        out_type=[
            jax.ShapeDtypeStruct((B, NP1), jnp.float32),
            jax.ShapeDtypeStruct((B, NP1), jnp.float32),
        ],
        mesh=mesh,
        compiler_params=pltpu.CompilerParams(needs_layout_passes=False, use_tc_tiling_on_sc=True),
        scratch_types=[
            pltpu.VMEM((NP1,), jnp.int32),
            pltpu.VMEM((J_CH, K), jnp.int32),
            pltpu.VMEM((J_CH, K), jnp.int32),
            pltpu.VMEM((NP1,), jnp.float32),
            pltpu.VMEM((NP1,), jnp.float32),
            pltpu.SemaphoreType.DMA,
            pltpu.SemaphoreType.DMA,
        ],
    )
    def sc_kernel(packed_hbm, knn_hbm, sx_hbm, sy_hbm,
                  table, idxa, idxb, sxb, syb, sema, semb):
        wid = lax.axis_index("s") * NC + lax.axis_index("c")
        iota = lax.broadcasted_iota(jnp.int32, (L,), 0)
        bufs = (idxa, idxb)
        sems = (sema, semb)

        def chunk_src(b, c):
            return knn_hbm.at[b, pl.ds(c * J_CH, J_CH)]

        for i in range(BPW):
            b = wid * BPW + i
            pltpu.sync_copy(packed_hbm.at[b], table)
            pltpu.async_copy(chunk_src(b, 0), bufs[0], sems[0])
            pltpu.async_copy(chunk_src(b, 1), bufs[1], sems[1])

            def chunk_pair(ci, _):
                for h in range(2):
                    c = ci * 2 + h
                    pltpu.make_async_copy(chunk_src(b, 0), bufs[h],
                                          sems[h]).wait()

                    @pl.when(c + 2 < N_CH)
                    def _():
                        pltpu.async_copy(chunk_src(b, c + 2), bufs[h], sems[h])

                    cur = bufs[h]

                    def body(g, _, c=c, cur=cur):
                        jl = g * L + iota
                        sxa = [jnp.zeros((L,), jnp.float32) for _ in range(4)]
                        sya = [jnp.zeros((L,), jnp.float32) for _ in range(4)]
                        for t in range(K):
                            rot = (iota + t) & (K - 1)
                            iv = plsc.load_gather(cur, [jl, rot])
                            w = plsc.load_gather(table, [iv])
                            x, y = plsc.unpack(plsc.bitcast(w, jnp.bfloat16),
                                               format=plsc.PackFormat.INTERLEAVED)
                            sxa[t % 4] = sxa[t % 4] + x
                            sya[t % 4] = sya[t % 4] + y
                        sx = (sxa[0] + sxa[1]) + (sxa[2] + sxa[3])
                        sy = (sya[0] + sya[1]) + (sya[2] + sya[3])
                        base = c * J_CH + g * L
                        sxb[pl.ds(base, L)] = sx
                        syb[pl.ds(base, L)] = sy
                        return 0

                    lax.fori_loop(0, J_CH // L, body, 0)
                return 0

            lax.fori_loop(0, N_CH // 2, chunk_pair, 0)
            pltpu.sync_copy(sxb, sx_hbm.at[b])
            pltpu.sync_copy(syb, sy_hbm.at[b])

    return sc_kernel


def _tc_body(lam_ref, qx_ref, qy_ref, packed_ref, sx_ref, sy_ref,
             mask_ref, o_ref):
    lam = lam_ref[0, 0]
    w = packed_ref[...]
    px = lax.bitcast_convert_type(w << 16, jnp.float32)
    py = lax.bitcast_convert_type(w & jnp.int32(-65536), jnp.float32)
    scores = (px * qx_ref[...] + py * qy_ref[...]
              + lam * (px * sx_ref[...] + py * sy_ref[...]))
    scores = jnp.where(mask_ref[...], jnp.float32(-1000000000.0), scores)
    m = jnp.max(scores, axis=1, keepdims=True)
    lse = m + jnp.log(jnp.sum(jnp.exp(scores - m), axis=1, keepdims=True))
    o_ref[...] = scores - lse


def _tc_score(lam, qx, qy, packed, sx, sy, mask):
    return pl.pallas_call(
        _tc_body,
        out_shape=jax.ShapeDtypeStruct((B, NP1), jnp.float32),
        in_specs=[
            pl.BlockSpec(memory_space=pltpu.SMEM),
            pl.BlockSpec(memory_space=pltpu.VMEM),
            pl.BlockSpec(memory_space=pltpu.VMEM),
            pl.BlockSpec(memory_space=pltpu.VMEM),
            pl.BlockSpec(memory_space=pltpu.VMEM),
            pl.BlockSpec(memory_space=pltpu.VMEM),
            pl.BlockSpec(memory_space=pltpu.VMEM),
        ],
    )(lam, qx, qy, packed, sx, sy, mask)


def kernel(query, psi_prime, knn_indices, mask, lambda_param):
    packed = lax.bitcast_convert_type(
        psi_prime.astype(jnp.bfloat16), jnp.int32)
    sx, sy = _sc_neighbor_sums()(packed, knn_indices)
    qx = query[:, 0:1]
    qy = query[:, 1:2]
    lam = jnp.reshape(lambda_param, (1, 1)).astype(jnp.float32)
    return _tc_score(lam, qx, qy, packed, sx, sy, mask)


# speedup vs baseline: 1.5448x; 1.5448x over previous
import functools

import numpy as np
import jax
import jax.numpy as jnp
from jax import lax
from jax.experimental import pallas as pl
from jax.experimental.pallas import tpu as pltpu
from jax.experimental.pallas import tpu_sc as plsc

B, NP1, K = 64, 4096, 32
NC, NS, L = 2, 16, 16
NW = NC * NS
BPW = B // NW
J_CH = 256
N_CH = NP1 // J_CH


@functools.lru_cache(maxsize=1)
def _sc_neighbor_sums():
    mesh = plsc.VectorSubcoreMesh(
        core_axis_name="c", subcore_axis_name="s", num_cores=NC, num_subcores=NS
    )

    @functools.partial(
        pl.kernel,
        out_type=[
            jax.ShapeDtypeStruct((B, NP1), jnp.float32),
            jax.ShapeDtypeStruct((B, NP1), jnp.float32),
        ],
        mesh=mesh,
        compiler_params=pltpu.CompilerParams(needs_layout_passes=False),
        scratch_types=[
            pltpu.VMEM((NP1,), jnp.int32),
            pltpu.VMEM((J_CH, K), jnp.int32),
            pltpu.VMEM((J_CH, K), jnp.int32),
            pltpu.VMEM((NP1,), jnp.float32),
            pltpu.VMEM((NP1,), jnp.float32),
            pltpu.SemaphoreType.DMA,
            pltpu.SemaphoreType.DMA,
        ],
    )
    def sc_kernel(packed_hbm, knn_hbm, sx_hbm, sy_hbm,
                  table, idxa, idxb, sxb, syb, sema, semb):
        wid = lax.axis_index("s") * NC + lax.axis_index("c")
        iota = lax.broadcasted_iota(jnp.int32, (L,), 0)
        bufs = (idxa, idxb)
        sems = (sema, semb)

        def chunk_src(b, c):
            return knn_hbm.at[pl.ds(b * NP1 + c * J_CH, J_CH)]

        for i in range(BPW):
            b = wid * BPW + i
            pltpu.sync_copy(packed_hbm.at[b], table)
            pltpu.async_copy(chunk_src(b, 0), bufs[0], sems[0])
            pltpu.async_copy(chunk_src(b, 1), bufs[1], sems[1])

            def chunk_pair(ci, _):
                for h in range(2):
                    c = ci * 2 + h
                    pltpu.make_async_copy(chunk_src(b, 0), bufs[h],
                                          sems[h]).wait()

                    @pl.when(c + 2 < N_CH)
                    def _():
                        pltpu.async_copy(chunk_src(b, c + 2), bufs[h], sems[h])

                    cur = bufs[h]

                    def body(g, _, c=c, cur=cur):
                        jl = g * L + iota
                        sxa = [jnp.zeros((L,), jnp.float32) for _ in range(4)]
                        sya = [jnp.zeros((L,), jnp.float32) for _ in range(4)]
                        for t in range(K):
                            rot = (iota + t) & (K - 1)
                            iv = plsc.load_gather(cur, [jl, rot])
                            w = plsc.load_gather(table, [iv])
                            x, y = plsc.unpack(plsc.bitcast(w, jnp.bfloat16),
                                               format=plsc.PackFormat.INTERLEAVED)
                            sxa[t % 4] = sxa[t % 4] + x
                            sya[t % 4] = sya[t % 4] + y
                        sx = (sxa[0] + sxa[1]) + (sxa[2] + sxa[3])
                        sy = (sya[0] + sya[1]) + (sya[2] + sya[3])
                        base = c * J_CH + g * L
                        sxb[pl.ds(base, L)] = sx
                        syb[pl.ds(base, L)] = sy
                        return 0

                    lax.fori_loop(0, J_CH // L, body, 0)
                return 0

            lax.fori_loop(0, N_CH // 2, chunk_pair, 0)
            pltpu.sync_copy(sxb, sx_hbm.at[b])
            pltpu.sync_copy(syb, sy_hbm.at[b])

    return sc_kernel


def _tc_body(lam_ref, qx_ref, qy_ref, packed_ref, sx_ref, sy_ref,
             mask_ref, o_ref):
    lam = lam_ref[0, 0]
    w = packed_ref[...]
    px = lax.bitcast_convert_type(w << 16, jnp.float32)
    py = lax.bitcast_convert_type(w & jnp.int32(-65536), jnp.float32)
    scores = (px * qx_ref[...] + py * qy_ref[...]
              + lam * (px * sx_ref[...] + py * sy_ref[...]))
    scores = jnp.where(mask_ref[...], jnp.float32(-1000000000.0), scores)
    m = jnp.max(scores, axis=1, keepdims=True)
    lse = m + jnp.log(jnp.sum(jnp.exp(scores - m), axis=1, keepdims=True))
    o_ref[...] = scores - lse


def _tc_score(lam, qx, qy, packed, sx, sy, mask):
    return pl.pallas_call(
        _tc_body,
        out_shape=jax.ShapeDtypeStruct((B, NP1), jnp.float32),
        in_specs=[
            pl.BlockSpec(memory_space=pltpu.SMEM),
            pl.BlockSpec(memory_space=pltpu.VMEM),
            pl.BlockSpec(memory_space=pltpu.VMEM),
            pl.BlockSpec(memory_space=pltpu.VMEM),
            pl.BlockSpec(memory_space=pltpu.VMEM),
            pl.BlockSpec(memory_space=pltpu.VMEM),
            pl.BlockSpec(memory_space=pltpu.VMEM),
        ],
    )(lam, qx, qy, packed, sx, sy, mask)


def kernel(query, psi_prime, knn_indices, mask, lambda_param):
    packed = lax.bitcast_convert_type(
        psi_prime.astype(jnp.bfloat16), jnp.int32)
    knn2d = jnp.reshape(knn_indices, (B * NP1, K))
    sx, sy = _sc_neighbor_sums()(packed, knn2d)
    qx = query[:, 0:1]
    qy = query[:, 1:2]
    lam = jnp.reshape(lambda_param, (1, 1)).astype(jnp.float32)
    return _tc_score(lam, qx, qy, packed, sx, sy, mask)
